# baseline (speedup 1.0000x reference)
import functools

import jax
import jax.numpy as jnp
from jax import lax
from jax.experimental import pallas as pl
from jax.experimental.pallas import tpu as pltpu
from jax.experimental.pallas import tpu_sc as plsc

B, C, H, W = 4096, 64, 4, 4
D = C * H * W
S = H * W
NQ = C // 32
PW = NQ * S
MAX_ID = 60000
NC, NS, L = 2, 16, 16
NW = NC * NS
BPW = B // NW
CH = 32
NCHUNK = BPW // CH

_mesh = plsc.VectorSubcoreMesh(core_axis_name="c", subcore_axis_name="s")


@functools.partial(
    pl.kernel,
    mesh=_mesh,
    out_type=jax.ShapeDtypeStruct((B, D), jnp.float32),
    scratch_types=[
        pltpu.VMEM((BPW,), jnp.int32),
        pltpu.VMEM((BPW, 128), jnp.int32),
        pltpu.VMEM((CH, D), jnp.float32),
        pltpu.SemaphoreType.DMA,
        pltpu.SemaphoreType.DMA,
    ],
)
def _tied_dropout(x_hbm, idx_hbm, packed_hbm, out_hbm,
                  idx_v, p_v, x_v, psem, xsem):
    wid = lax.axis_index("s") * NC + lax.axis_index("c")
    base = wid * BPW
    pltpu.sync_copy(idx_hbm.at[pl.ds(base, BPW)], idx_v)
    pc = pltpu.async_copy(packed_hbm.at[idx_v], p_v, psem)
    pc.wait()
    for k in range(NCHUNK):
        row0 = base + k * CH
        xc = pltpu.async_copy(x_hbm.at[pl.ds(row0, CH)], x_v, xsem)
        xc.wait()

        def row_body(r, _):
            p0 = p_v[k * CH + r, pl.ds(0, L)]
            p1 = p_v[k * CH + r, pl.ds(L, L)]

            def bit_body(j, _):
                b0 = ((p0 >> j) & 1).astype(jnp.float32)
                b1 = ((p1 >> j) & 1).astype(jnp.float32)
                c0 = j * L
                c1 = (32 + j) * L
                x_v[r, pl.ds(c0, L)] = x_v[r, pl.ds(c0, L)] * b0
                x_v[r, pl.ds(c1, L)] = x_v[r, pl.ds(c1, L)] * b1
                return 0

            lax.fori_loop(0, 32, bit_body, 0)
            return 0

        lax.fori_loop(0, CH, row_body, 0)
        pltpu.sync_copy(x_v, out_hbm.at[pl.ds(row0, CH)])


def kernel(X, idx, mask_tensor):
    bits = (mask_tensor != 0).astype(jnp.int32)
    shifts = jnp.arange(32, dtype=jnp.int32).reshape(1, 1, 32, 1)
    packed = jnp.sum(bits.reshape(MAX_ID, NQ, 32, S) << shifts, axis=2,
                     dtype=jnp.int32)
    packed = jnp.pad(packed.reshape(MAX_ID, PW), ((0, 0), (0, 128 - PW)))
    x2 = X.reshape(B, D)
    out = _tied_dropout(x2, idx, packed)
    return out.reshape(B, C, H, W)
